# Initial kernel scaffold; baseline (speedup 1.0000x reference)
import jax
import jax.numpy as jnp
from jax.experimental import pallas as pl


def kernel(x, router_w, W1, W2, W3):
    raise NotImplementedError("write your pallas kernel here")


# speedup vs baseline: 1.6802x; 1.6802x over previous
import jax
import jax.numpy as jnp
from jax.experimental import pallas as pl

D_MODEL = 768
NUM_EXPERTS = 8
D_FF = 2048
TOP_K = 2

_T_TILE = 512


def _dot_t(a, b):
    return jax.lax.dot_general(
        a, b, dimension_numbers=(((1,), (1,)), ((), ())),
        preferred_element_type=jnp.float32)


def _moe_body(x_ref, rw_ref, w1_ref, w2_ref, w3_ref, o_ref):
    e = pl.program_id(1)
    xt = x_ref[...]
    logits = _dot_t(xt, rw_ref[...])
    ii = jax.lax.broadcasted_iota(jnp.int32, logits.shape, 1)
    m1 = jnp.max(logits, axis=1, keepdims=True)
    i1 = jnp.min(jnp.where(logits == m1, ii, NUM_EXPERTS), axis=1, keepdims=True)
    l2 = jnp.where(ii == i1, -jnp.inf, logits)
    m2 = jnp.max(l2, axis=1, keepdims=True)
    i2 = jnp.min(jnp.where(l2 == m2, ii, NUM_EXPERTS), axis=1, keepdims=True)
    r = jnp.exp(m2 - m1)
    p1 = 1.0 / (1.0 + r)
    p2 = r * p1
    w_e = p1 * (i1 == e) + p2 * (i2 == e)

    a = _dot_t(xt, w1_ref[0])
    b = _dot_t(xt, w3_ref[0])
    h = (a * jax.nn.sigmoid(a)) * b
    y = _dot_t(h, w2_ref[0])
    contrib = w_e * y

    @pl.when(e == 0)
    def _init():
        o_ref[...] = contrib

    @pl.when(e != 0)
    def _acc():
        o_ref[...] += contrib


def kernel(x, router_w, W1, W2, W3):
    B, S, D = x.shape
    T = B * S
    xt = x.reshape(T, D)
    n_tiles = T // _T_TILE
    out = pl.pallas_call(
        _moe_body,
        grid=(n_tiles, NUM_EXPERTS),
        in_specs=[
            pl.BlockSpec((_T_TILE, D), lambda t, e: (t, 0)),
            pl.BlockSpec((NUM_EXPERTS, D), lambda t, e: (0, 0)),
            pl.BlockSpec((1, D_FF, D), lambda t, e: (e, 0, 0)),
            pl.BlockSpec((1, D, D_FF), lambda t, e: (e, 0, 0)),
            pl.BlockSpec((1, D_FF, D), lambda t, e: (e, 0, 0)),
        ],
        out_specs=pl.BlockSpec((_T_TILE, D), lambda t, e: (t, 0)),
        out_shape=jax.ShapeDtypeStruct((T, D), jnp.float32),
    )(xt, router_w, W1, W2, W3)
    return out.reshape(B, S, D)
